# baseline (speedup 1.0000x reference)
import functools

import jax
import jax.numpy as jnp
from jax.experimental import pallas as pl
from jax.experimental.pallas import tpu as pltpu

TOP_K = 8
ROUTED_SCALING_FACTOR = 2.5


def _gate_block(x_ref, w_ref, b_ref, idx_ref, wgt_ref):
    x = x_ref[...]
    w = w_ref[...]
    logits = jax.lax.dot_general(
        w, x, (((1,), (1,)), ((), ())), preferred_element_type=jnp.float32
    )
    scores = jax.nn.sigmoid(logits)
    biased = scores + b_ref[...]

    e, bt = scores.shape
    eidx = jax.lax.broadcasted_iota(jnp.int32, (e, bt), 0).astype(jnp.float32)
    pay = scores + eidx

    work = biased
    ssum = jnp.zeros((1, bt), jnp.float32)
    picked = []
    for k in range(TOP_K):
        mx = jnp.max(work, axis=0, keepdims=True)
        paym = jnp.min(jnp.where(work == mx, pay, 3.0e38), axis=0, keepdims=True)
        idxf = jnp.floor(paym)
        sk = paym - idxf
        picked.append(paym)
        ssum = ssum + sk
        if k + 1 < TOP_K:
            work = jnp.where(pay == paym, -jnp.inf, work)

    inv = ROUTED_SCALING_FACTOR / (ssum + 1e-20)
    paystack = jnp.concatenate(picked, axis=0).T
    idxs = jnp.floor(paystack)
    idx_ref[...] = idxs.astype(jnp.int32)
    wgt_ref[...] = (paystack - idxs) * inv.T


@functools.partial(jax.jit, static_argnames=("block_t",))
def _route(x, gate_weight, bias2d, block_t=4096):
    t, d = x.shape
    e = gate_weight.shape[0]
    grid = (t // block_t,)
    idx_t, wgt_t = pl.pallas_call(
        _gate_block,
        grid=grid,
        in_specs=[
            pl.BlockSpec((block_t, d), lambda i: (i, 0)),
            pl.BlockSpec((e, d), lambda i: (0, 0)),
            pl.BlockSpec((e, 1), lambda i: (0, 0)),
        ],
        out_specs=[
            pl.BlockSpec((block_t, TOP_K), lambda i: (i, 0)),
            pl.BlockSpec((block_t, TOP_K), lambda i: (i, 0)),
        ],
        out_shape=[
            jax.ShapeDtypeStruct((t, TOP_K), jnp.int32),
            jax.ShapeDtypeStruct((t, TOP_K), jnp.float32),
        ],
    )(x, gate_weight, bias2d)
    return idx_t, wgt_t


def kernel(x, gate_weight, e_score_correction_bias):
    x = x.astype(jnp.float32)
    w = gate_weight.astype(jnp.float32)
    b = e_score_correction_bias.astype(jnp.float32).reshape(-1, 1)
    idx, wgt = _route(x, w, b)
    return idx, wgt


# speedup vs baseline: 1.6743x; 1.6743x over previous
import functools

import jax
import jax.numpy as jnp
from jax.experimental import pallas as pl
from jax.experimental.pallas import tpu as pltpu

TOP_K = 8
ROUTED_SCALING_FACTOR = 2.5


def _gate_block(x_ref, w_ref, b_ref, idx_ref, wgt_ref):
    x = x_ref[...]
    w = w_ref[...]
    logits = jax.lax.dot_general(
        w, x, (((1,), (1,)), ((), ())), preferred_element_type=jnp.float32
    )
    scores = jax.nn.sigmoid(logits)
    biased = scores + b_ref[...]

    e, bt = scores.shape
    eidx = jax.lax.broadcasted_iota(jnp.int32, (e, bt), 0).astype(jnp.float32)
    pay = scores + eidx

    work = biased
    ssum = jnp.zeros((1, bt), jnp.float32)
    picked = []
    for k in range(TOP_K):
        mx = jnp.max(work, axis=0, keepdims=True)
        paym = jnp.min(jnp.where(work == mx, pay, 3.0e38), axis=0, keepdims=True)
        idxf = jnp.floor(paym)
        sk = paym - idxf
        idx_ref[k : k + 1, :] = idxf.astype(jnp.int32)
        picked.append(sk)
        ssum = ssum + sk
        if k + 1 < TOP_K:
            work = jnp.where(pay == paym, -jnp.inf, work)

    inv = ROUTED_SCALING_FACTOR / (ssum + 1e-20)
    wgt_ref[...] = jnp.concatenate(picked, axis=0) * inv


@functools.partial(jax.jit, static_argnames=("block_t",))
def _route(x, gate_weight, bias2d, block_t=4096):
    t, d = x.shape
    e = gate_weight.shape[0]
    grid = (t // block_t,)
    idx_t, wgt_t = pl.pallas_call(
        _gate_block,
        grid=grid,
        in_specs=[
            pl.BlockSpec((block_t, d), lambda i: (i, 0)),
            pl.BlockSpec((e, d), lambda i: (0, 0)),
            pl.BlockSpec((e, 1), lambda i: (0, 0)),
        ],
        out_specs=[
            pl.BlockSpec((TOP_K, block_t), lambda i: (0, i)),
            pl.BlockSpec((TOP_K, block_t), lambda i: (0, i)),
        ],
        out_shape=[
            jax.ShapeDtypeStruct((TOP_K, t), jnp.int32),
            jax.ShapeDtypeStruct((TOP_K, t), jnp.float32),
        ],
    )(x, gate_weight, bias2d)
    return idx_t.T, wgt_t.T


def kernel(x, gate_weight, e_score_correction_bias):
    x = x.astype(jnp.float32)
    w = gate_weight.astype(jnp.float32)
    b = e_score_correction_bias.astype(jnp.float32).reshape(-1, 1)
    idx, wgt = _route(x, w, b)
    return idx, wgt
